# baseline (speedup 1.0000x reference)
import functools

import jax
import jax.numpy as jnp
from jax import lax
from jax.experimental import pallas as pl
from jax.experimental.pallas import tpu as pltpu
from jax.experimental.pallas import tpu_sc as plsc


def _sc_gather_pairs(user_wide, item_wide, uidx_w, iidx_w, batch):
    info = plsc.get_sparse_core_info()
    nc, ns = info.num_cores, info.num_subcores
    nw = nc * ns
    b_per_w = batch // nw
    chunk = b_per_w // 2
    width = user_wide.shape[1]
    mesh = plsc.VectorSubcoreMesh(core_axis_name="c", subcore_axis_name="s")

    @functools.partial(
        pl.kernel,
        mesh=mesh,
        out_type=(
            jax.ShapeDtypeStruct((batch, width), jnp.float32),
            jax.ShapeDtypeStruct((batch, width), jnp.float32),
        ),
        scratch_types=[
            pltpu.VMEM((chunk,), jnp.int32),
            pltpu.VMEM((chunk, width), jnp.float32),
            pltpu.VMEM((chunk,), jnp.int32),
            pltpu.VMEM((chunk, width), jnp.float32),
            pltpu.SemaphoreType.DMA,
            pltpu.SemaphoreType.DMA,
        ],
    )
    def gather_kernel(ut_hbm, it_hbm, ui_hbm, ii_hbm, uw_hbm, iw_hbm,
                      uidx_v, urows_v, iidx_v, irows_v, usem, isem):
        wid = lax.axis_index("s") * nc + lax.axis_index("c")
        tile_base = wid * b_per_w
        for c in range(2):
            base = tile_base + c * chunk
            pltpu.sync_copy(ui_hbm.at[pl.ds(base, chunk)], uidx_v)
            pltpu.sync_copy(ii_hbm.at[pl.ds(base, chunk)], iidx_v)
            ucopy = pltpu.async_copy(ut_hbm.at[uidx_v], urows_v, usem)
            icopy = pltpu.async_copy(it_hbm.at[iidx_v], irows_v, isem)
            ucopy.wait()
            icopy.wait()
            pltpu.sync_copy(urows_v, uw_hbm.at[pl.ds(base, chunk)])
            pltpu.sync_copy(irows_v, iw_hbm.at[pl.ds(base, chunk)])

    return gather_kernel(user_wide, item_wide, uidx_w, iidx_w)


def _mlp_body(uw_ref, iw_ref, up_ref, ip_ref, low_ref, w1u_ref, w1i_ref,
              b1_ref, w2_ref, b2_ref, out_ref):
    low = low_ref[...]
    flip = 1.0 - 2.0 * low
    umask = low + up_ref[...] * flip
    imask = low + ip_ref[...] * flip
    uwm = uw_ref[...] * umask
    iwm = iw_ref[...] * imask
    h = (
        jnp.dot(uwm, w1u_ref[...], preferred_element_type=jnp.float32)
        + jnp.dot(iwm, w1i_ref[...], preferred_element_type=jnp.float32)
        + b1_ref[...]
    )
    h = jnp.maximum(h, 0.0)
    out_ref[...] = jnp.sum(h * w2_ref[...], axis=1, keepdims=True) + b2_ref[...]


def _tc_mlp(uw, iw, up, ip, W1, b1, W2, b2, batch, block):
    width = uw.shape[1]
    half = width // 2
    hidden = W1.shape[1]
    w1u = jnp.concatenate([W1[:half], W1[:half]], axis=0)
    w1i = jnp.concatenate([W1[half:], W1[half:]], axis=0)
    b1r = b1.reshape(1, hidden)
    w2r = W2.reshape(1, hidden)
    b2r = b2.reshape(1, 1)
    low = jnp.concatenate(
        [jnp.ones((1, half), jnp.float32), jnp.zeros((1, half), jnp.float32)],
        axis=1,
    )
    grid = (batch // block,)
    return pl.pallas_call(
        _mlp_body,
        grid=grid,
        in_specs=[
            pl.BlockSpec((block, width), lambda i: (i, 0)),
            pl.BlockSpec((block, width), lambda i: (i, 0)),
            pl.BlockSpec((block, 1), lambda i: (i, 0)),
            pl.BlockSpec((block, 1), lambda i: (i, 0)),
            pl.BlockSpec((1, width), lambda i: (0, 0)),
            pl.BlockSpec((width, hidden), lambda i: (0, 0)),
            pl.BlockSpec((width, hidden), lambda i: (0, 0)),
            pl.BlockSpec((1, hidden), lambda i: (0, 0)),
            pl.BlockSpec((1, hidden), lambda i: (0, 0)),
            pl.BlockSpec((1, 1), lambda i: (0, 0)),
        ],
        out_specs=pl.BlockSpec((block, 1), lambda i: (i, 0)),
        out_shape=jax.ShapeDtypeStruct((batch, 1), jnp.float32),
    )(uw, iw, up, ip, low, w1u, w1i, b1r, w2r, b2r)


@jax.jit
def kernel(user_idx, item_idx, user_table, item_table, W1, b1, W2, b2):
    batch = user_idx.shape[0]
    embed = user_table.shape[1]
    ui = user_idx.astype(jnp.int32)
    ii = item_idx.astype(jnp.int32)
    user_wide = user_table.reshape(-1, 2 * embed)
    item_wide = item_table.reshape(-1, 2 * embed)
    uw, iw = _sc_gather_pairs(
        user_wide, item_wide,
        lax.shift_right_logical(ui, 1), lax.shift_right_logical(ii, 1), batch,
    )
    up = lax.bitwise_and(ui, 1).astype(jnp.float32).reshape(batch, 1)
    ip = lax.bitwise_and(ii, 1).astype(jnp.float32).reshape(batch, 1)
    return _tc_mlp(uw, iw, up, ip, W1, b1, W2, b2, batch, block=2048)


# speedup vs baseline: 1.6240x; 1.6240x over previous
import functools

import jax
import jax.numpy as jnp
from jax import lax
from jax.experimental import pallas as pl
from jax.experimental.pallas import tpu as pltpu
from jax.experimental.pallas import tpu_sc as plsc


def _sc_gather(user_table, item_table, user_idx, item_idx, batch):
    info = plsc.get_sparse_core_info()
    nc, ns = info.num_cores, info.num_subcores
    nw = nc * ns
    b_per_w = batch // nw
    embed = user_table.shape[1]
    mesh = plsc.VectorSubcoreMesh(core_axis_name="c", subcore_axis_name="s")

    @functools.partial(
        pl.kernel,
        mesh=mesh,
        out_type=(
            jax.ShapeDtypeStruct((batch, embed), jnp.float32),
            jax.ShapeDtypeStruct((batch, embed), jnp.float32),
        ),
        scratch_types=[
            pltpu.VMEM((b_per_w,), jnp.int32),
            pltpu.VMEM((b_per_w,), jnp.int32),
            pltpu.VMEM((b_per_w // 2, embed), jnp.float32),
            pltpu.VMEM((b_per_w // 2, embed), jnp.float32),
            pltpu.SemaphoreType.DMA,
            pltpu.SemaphoreType.DMA,
        ],
    )
    def gather_kernel(ut_hbm, it_hbm, ui_hbm, ii_hbm, ue_hbm, ie_hbm,
                      uidx_s, iidx_s, urows_v, irows_v, usem, isem):
        chunk = b_per_w // 2
        wid = lax.axis_index("s") * nc + lax.axis_index("c")
        base = wid * b_per_w
        pltpu.sync_copy(ui_hbm.at[pl.ds(base, b_per_w)], uidx_s)
        pltpu.sync_copy(ii_hbm.at[pl.ds(base, b_per_w)], iidx_s)

        k = 16
        n_g = chunk // k

        for c in range(2):
            cb = c * chunk

            def issue_group(g, cb=cb):
                gb = g * k
                uvec = uidx_s[pl.ds(cb + gb, k)]
                ivec = iidx_s[pl.ds(cb + gb, k)]
                for j in range(k):
                    pltpu.async_copy(ut_hbm.at[pl.ds(uvec[j], 1)],
                                     urows_v.at[pl.ds(gb + j, 1)], usem)
                    pltpu.async_copy(it_hbm.at[pl.ds(ivec[j], 1)],
                                     irows_v.at[pl.ds(gb + j, 1)], isem)

            def drain_group(g):
                gb = g * k
                for j in range(k):
                    pltpu.make_async_copy(ut_hbm.at[pl.ds(0, 1)],
                                          urows_v.at[pl.ds(gb + j, 1)],
                                          usem).wait()
                    pltpu.make_async_copy(it_hbm.at[pl.ds(0, 1)],
                                          irows_v.at[pl.ds(gb + j, 1)],
                                          isem).wait()

            issue_group(0)

            @pl.loop(1, n_g)
            def _(g):
                issue_group(g)
                drain_group(g - 1)

            drain_group(n_g - 1)
            pltpu.sync_copy(urows_v, ue_hbm.at[pl.ds(base + cb, chunk)])
            pltpu.sync_copy(irows_v, ie_hbm.at[pl.ds(base + cb, chunk)])

    return gather_kernel(user_table, item_table, user_idx, item_idx)


def _mlp_body(ue_ref, ie_ref, w1u_ref, w1i_ref, b1_ref, w2_ref, b2_ref,
              out_ref):
    h = (
        jnp.dot(ue_ref[...], w1u_ref[...], preferred_element_type=jnp.float32)
        + jnp.dot(ie_ref[...], w1i_ref[...], preferred_element_type=jnp.float32)
        + b1_ref[...]
    )
    h = jnp.maximum(h, 0.0)
    out_ref[...] = jnp.sum(h * w2_ref[...], axis=1, keepdims=True) + b2_ref[...]


def _tc_mlp(ue, ie, W1, b1, W2, b2, batch, block):
    embed = ue.shape[1]
    hidden = W1.shape[1]
    w1u = W1[:embed]
    w1i = W1[embed:]
    b1r = b1.reshape(1, hidden)
    w2r = W2.reshape(1, hidden)
    b2r = b2.reshape(1, 1)
    grid = (batch // block,)
    return pl.pallas_call(
        _mlp_body,
        grid=grid,
        in_specs=[
            pl.BlockSpec((block, embed), lambda i: (i, 0)),
            pl.BlockSpec((block, embed), lambda i: (i, 0)),
            pl.BlockSpec((embed, hidden), lambda i: (0, 0)),
            pl.BlockSpec((embed, hidden), lambda i: (0, 0)),
            pl.BlockSpec((1, hidden), lambda i: (0, 0)),
            pl.BlockSpec((1, hidden), lambda i: (0, 0)),
            pl.BlockSpec((1, 1), lambda i: (0, 0)),
        ],
        out_specs=pl.BlockSpec((block, 1), lambda i: (i, 0)),
        out_shape=jax.ShapeDtypeStruct((batch, 1), jnp.float32),
    )(ue, ie, w1u, w1i, b1r, w2r, b2r)


@jax.jit
def kernel(user_idx, item_idx, user_table, item_table, W1, b1, W2, b2):
    batch = user_idx.shape[0]
    ue, ie = _sc_gather(
        user_table, item_table,
        user_idx.astype(jnp.int32), item_idx.astype(jnp.int32), batch,
    )
    return _tc_mlp(ue, ie, W1, b1, W2, b2, batch, block=2048)


# speedup vs baseline: 1.6708x; 1.0289x over previous
import functools

import jax
import jax.numpy as jnp
from jax import lax
from jax.experimental import pallas as pl
from jax.experimental.pallas import tpu as pltpu
from jax.experimental.pallas import tpu_sc as plsc

_COLS = 4096


def _repack_body(in_ref, out_ref):
    t = jnp.transpose(in_ref[...])
    half = t.shape[0] // 2
    out_ref[...] = jnp.concatenate([t[:half], t[half:]], axis=1)


def _tc_repack_pairs(tbl_t):
    embed = tbl_t.shape[0]
    n = tbl_t.shape[1]
    n_blocks = pl.cdiv(n, _COLS)
    return pl.pallas_call(
        _repack_body,
        grid=(n_blocks,),
        in_specs=[pl.BlockSpec((embed, _COLS), lambda j: (0, j))],
        out_specs=pl.BlockSpec((_COLS // 2, 2 * embed), lambda j: (j, 0)),
        out_shape=jax.ShapeDtypeStruct(
            (n_blocks * _COLS // 2, 2 * embed), jnp.float32),
    )(tbl_t)


def _pair_index(r):
    half_cols = _COLS // 2
    pidx = (r // _COLS) * half_cols + lax.rem(r, half_cols)
    p = lax.rem(r // half_cols, 2)
    return pidx, p


def _sc_pair_gather(pairs, pidx, batch):
    info = plsc.get_sparse_core_info()
    nc, ns = info.num_cores, info.num_subcores
    nw = nc * ns
    b_per_w = batch // nw
    chunk = b_per_w // 2
    width = pairs.shape[1]
    mesh = plsc.VectorSubcoreMesh(core_axis_name="c", subcore_axis_name="s")

    @functools.partial(
        pl.kernel,
        mesh=mesh,
        out_type=jax.ShapeDtypeStruct((batch, width), jnp.float32),
        scratch_types=[
            pltpu.VMEM((chunk,), jnp.int32),
            pltpu.VMEM((chunk, width), jnp.float32),
            pltpu.SemaphoreType.DMA,
        ],
    )
    def gather_kernel(tbl_hbm, idx_hbm, out_hbm, idx_v, rows_v, sem):
        wid = lax.axis_index("s") * nc + lax.axis_index("c")
        tile_base = wid * b_per_w
        for c in range(b_per_w // chunk):
            base = tile_base + c * chunk
            pltpu.sync_copy(idx_hbm.at[pl.ds(base, chunk)], idx_v)
            pltpu.async_copy(tbl_hbm.at[idx_v], rows_v, sem).wait()
            pltpu.sync_copy(rows_v, out_hbm.at[pl.ds(base, chunk)])

    return gather_kernel(pairs, pidx)


def _mlp_body(uw_ref, iw_ref, up_ref, ip_ref, low_ref, w1u_ref, w1i_ref,
              b1_ref, w2_ref, b2_ref, out_ref):
    low = low_ref[...]
    flip = 1.0 - 2.0 * low
    umask = low + up_ref[...] * flip
    imask = low + ip_ref[...] * flip
    uwm = uw_ref[...] * umask
    iwm = iw_ref[...] * imask
    h = (
        jnp.dot(uwm, w1u_ref[...], preferred_element_type=jnp.float32)
        + jnp.dot(iwm, w1i_ref[...], preferred_element_type=jnp.float32)
        + b1_ref[...]
    )
    h = jnp.maximum(h, 0.0)
    out_ref[...] = jnp.sum(h * w2_ref[...], axis=1, keepdims=True) + b2_ref[...]


def _tc_mlp(uw, iw, up, ip, W1, b1, W2, b2, batch, block):
    width = uw.shape[1]
    half = width // 2
    hidden = W1.shape[1]
    w1u = jnp.concatenate([W1[:half], W1[:half]], axis=0)
    w1i = jnp.concatenate([W1[half:], W1[half:]], axis=0)
    b1r = b1.reshape(1, hidden)
    w2r = W2.reshape(1, hidden)
    b2r = b2.reshape(1, 1)
    low = jnp.concatenate(
        [jnp.ones((1, half), jnp.float32), jnp.zeros((1, half), jnp.float32)],
        axis=1,
    )
    grid = (batch // block,)
    return pl.pallas_call(
        _mlp_body,
        grid=grid,
        in_specs=[
            pl.BlockSpec((block, width), lambda i: (i, 0)),
            pl.BlockSpec((block, width), lambda i: (i, 0)),
            pl.BlockSpec((block, 1), lambda i: (i, 0)),
            pl.BlockSpec((block, 1), lambda i: (i, 0)),
            pl.BlockSpec((1, width), lambda i: (0, 0)),
            pl.BlockSpec((width, hidden), lambda i: (0, 0)),
            pl.BlockSpec((width, hidden), lambda i: (0, 0)),
            pl.BlockSpec((1, hidden), lambda i: (0, 0)),
            pl.BlockSpec((1, hidden), lambda i: (0, 0)),
            pl.BlockSpec((1, 1), lambda i: (0, 0)),
        ],
        out_specs=pl.BlockSpec((block, 1), lambda i: (i, 0)),
        out_shape=jax.ShapeDtypeStruct((batch, 1), jnp.float32),
    )(uw, iw, up, ip, low, w1u, w1i, b1r, w2r, b2r)


@jax.jit
def kernel(user_idx, item_idx, user_table, item_table, W1, b1, W2, b2):
    batch = user_idx.shape[0]
    ui = user_idx.astype(jnp.int32)
    ii = item_idx.astype(jnp.int32)
    upidx, up = _pair_index(ui)
    ipidx, ip = _pair_index(ii)
    item_pairs = _tc_repack_pairs(item_table.T)
    iw = _sc_pair_gather(item_pairs, ipidx, batch)
    user_pairs = _tc_repack_pairs(user_table.T)
    uw = _sc_pair_gather(user_pairs, upidx, batch)
    upf = up.astype(jnp.float32).reshape(batch, 1)
    ipf = ip.astype(jnp.float32).reshape(batch, 1)
    return _tc_mlp(uw, iw, upf, ipf, W1, b1, W2, b2, batch, block=2048)


# speedup vs baseline: 1.6712x; 1.0002x over previous
import functools

import jax
import jax.numpy as jnp
from jax import lax
from jax.experimental import pallas as pl
from jax.experimental.pallas import tpu as pltpu
from jax.experimental.pallas import tpu_sc as plsc

_COLS = 4096


def _repack_body(in_ref, ident_ref, out_ref):
    t = lax.dot_general(in_ref[...], ident_ref[...],
                        (((0,), (0,)), ((), ())),
                        preferred_element_type=jnp.float32)
    half = t.shape[0] // 2
    out_ref[...] = jnp.concatenate([t[:half], t[half:]], axis=1)


def _tc_repack_pairs(tbl_t):
    embed = tbl_t.shape[0]
    n = tbl_t.shape[1]
    n_blocks = pl.cdiv(n, _COLS)
    ident = jnp.eye(embed, dtype=jnp.float32)
    return pl.pallas_call(
        _repack_body,
        grid=(n_blocks,),
        compiler_params=pltpu.CompilerParams(
            dimension_semantics=("parallel",)),
        in_specs=[pl.BlockSpec((embed, _COLS), lambda j: (0, j)),
                  pl.BlockSpec((embed, embed), lambda j: (0, 0))],
        out_specs=pl.BlockSpec((_COLS // 2, 2 * embed), lambda j: (j, 0)),
        out_shape=jax.ShapeDtypeStruct(
            (n_blocks * _COLS // 2, 2 * embed), jnp.float32),
    )(tbl_t, ident)


def _pair_index(r):
    half_cols = _COLS // 2
    pidx = (r // _COLS) * half_cols + lax.rem(r, half_cols)
    p = lax.rem(r // half_cols, 2)
    return pidx, p


def _sc_pair_gather(pairs, pidx, batch):
    info = plsc.get_sparse_core_info()
    nc, ns = info.num_cores, info.num_subcores
    nw = nc * ns
    b_per_w = batch // nw
    chunk = b_per_w // 2
    width = pairs.shape[1]
    mesh = plsc.VectorSubcoreMesh(core_axis_name="c", subcore_axis_name="s")

    @functools.partial(
        pl.kernel,
        mesh=mesh,
        out_type=jax.ShapeDtypeStruct((batch, width), jnp.float32),
        scratch_types=[
            pltpu.VMEM((chunk,), jnp.int32),
            pltpu.VMEM((chunk, width), jnp.float32),
            pltpu.SemaphoreType.DMA,
        ],
    )
    def gather_kernel(tbl_hbm, idx_hbm, out_hbm, idx_v, rows_v, sem):
        wid = lax.axis_index("s") * nc + lax.axis_index("c")
        tile_base = wid * b_per_w
        for c in range(b_per_w // chunk):
            base = tile_base + c * chunk
            pltpu.sync_copy(idx_hbm.at[pl.ds(base, chunk)], idx_v)
            pltpu.async_copy(tbl_hbm.at[idx_v], rows_v, sem).wait()
            pltpu.sync_copy(rows_v, out_hbm.at[pl.ds(base, chunk)])

    return gather_kernel(pairs, pidx)


def _mlp_body(uw_ref, iw_ref, up_ref, ip_ref, low_ref, w1u_ref, w1i_ref,
              b1_ref, w2_ref, b2_ref, out_ref):
    low = low_ref[...]
    flip = 1.0 - 2.0 * low
    umask = low + up_ref[...] * flip
    imask = low + ip_ref[...] * flip
    uwm = uw_ref[...] * umask
    iwm = iw_ref[...] * imask
    h = (
        jnp.dot(uwm, w1u_ref[...], preferred_element_type=jnp.float32)
        + jnp.dot(iwm, w1i_ref[...], preferred_element_type=jnp.float32)
        + b1_ref[...]
    )
    h = jnp.maximum(h, 0.0)
    out_ref[...] = jnp.sum(h * w2_ref[...], axis=1, keepdims=True) + b2_ref[...]


def _tc_mlp(uw, iw, up, ip, W1, b1, W2, b2, batch, block):
    width = uw.shape[1]
    half = width // 2
    hidden = W1.shape[1]
    w1u = jnp.concatenate([W1[:half], W1[:half]], axis=0)
    w1i = jnp.concatenate([W1[half:], W1[half:]], axis=0)
    b1r = b1.reshape(1, hidden)
    w2r = W2.reshape(1, hidden)
    b2r = b2.reshape(1, 1)
    low = jnp.concatenate(
        [jnp.ones((1, half), jnp.float32), jnp.zeros((1, half), jnp.float32)],
        axis=1,
    )
    grid = (batch // block,)
    return pl.pallas_call(
        _mlp_body,
        grid=grid,
        in_specs=[
            pl.BlockSpec((block, width), lambda i: (i, 0)),
            pl.BlockSpec((block, width), lambda i: (i, 0)),
            pl.BlockSpec((block, 1), lambda i: (i, 0)),
            pl.BlockSpec((block, 1), lambda i: (i, 0)),
            pl.BlockSpec((1, width), lambda i: (0, 0)),
            pl.BlockSpec((width, hidden), lambda i: (0, 0)),
            pl.BlockSpec((width, hidden), lambda i: (0, 0)),
            pl.BlockSpec((1, hidden), lambda i: (0, 0)),
            pl.BlockSpec((1, hidden), lambda i: (0, 0)),
            pl.BlockSpec((1, 1), lambda i: (0, 0)),
        ],
        out_specs=pl.BlockSpec((block, 1), lambda i: (i, 0)),
        out_shape=jax.ShapeDtypeStruct((batch, 1), jnp.float32),
    )(uw, iw, up, ip, low, w1u, w1i, b1r, w2r, b2r)


@jax.jit
def kernel(user_idx, item_idx, user_table, item_table, W1, b1, W2, b2):
    batch = user_idx.shape[0]
    ui = user_idx.astype(jnp.int32)
    ii = item_idx.astype(jnp.int32)
    upidx, up = _pair_index(ui)
    ipidx, ip = _pair_index(ii)
    item_pairs = _tc_repack_pairs(item_table.T)
    iw = _sc_pair_gather(item_pairs, ipidx, batch)
    user_pairs = _tc_repack_pairs(user_table.T)
    uw = _sc_pair_gather(user_pairs, upidx, batch)
    upf = up.astype(jnp.float32).reshape(batch, 1)
    ipf = ip.astype(jnp.float32).reshape(batch, 1)
    return _tc_mlp(uw, iw, upf, ipf, W1, b1, W2, b2, batch, block=2048)


# speedup vs baseline: 2.2450x; 1.3433x over previous
import functools

import jax
import jax.numpy as jnp
from jax import lax
from jax.experimental import pallas as pl
from jax.experimental.pallas import tpu as pltpu
from jax.experimental.pallas import tpu_sc as plsc

_COLS = 16384


def _repack_body(in_ref, ident_ref, out_ref):
    t = lax.dot_general(in_ref[...], ident_ref[...],
                        (((0,), (0,)), ((), ())),
                        preferred_element_type=jnp.float32)
    half = t.shape[0] // 2
    out_ref[...] = jnp.concatenate([t[:half], t[half:]], axis=1)


def _tc_repack_pairs(tbl_t):
    embed = tbl_t.shape[0]
    n = tbl_t.shape[1]
    n_blocks = pl.cdiv(n, _COLS)
    ident = jnp.eye(embed, dtype=jnp.float32)
    return pl.pallas_call(
        _repack_body,
        grid=(n_blocks,),
        compiler_params=pltpu.CompilerParams(
            dimension_semantics=("parallel",)),
        in_specs=[pl.BlockSpec((embed, _COLS), lambda j: (0, j)),
                  pl.BlockSpec((embed, embed), lambda j: (0, 0))],
        out_specs=pl.BlockSpec((_COLS // 2, 2 * embed), lambda j: (j, 0)),
        out_shape=jax.ShapeDtypeStruct(
            (n_blocks * _COLS // 2, 2 * embed), jnp.float32),
    )(tbl_t, ident)


def _pair_index(r):
    half_cols = _COLS // 2
    pidx = (r // _COLS) * half_cols + lax.rem(r, half_cols)
    p = lax.rem(r // half_cols, 2)
    return pidx, p


def _sc_pair_gather(pairs, pidx, batch):
    info = plsc.get_sparse_core_info()
    nc, ns = info.num_cores, info.num_subcores
    nw = nc * ns
    b_per_w = batch // nw
    chunk = b_per_w // 2
    width = pairs.shape[1]
    mesh = plsc.VectorSubcoreMesh(core_axis_name="c", subcore_axis_name="s")

    @functools.partial(
        pl.kernel,
        mesh=mesh,
        out_type=jax.ShapeDtypeStruct((batch, width), jnp.float32),
        scratch_types=[
            pltpu.VMEM((chunk,), jnp.int32),
            pltpu.VMEM((chunk, width), jnp.float32),
            pltpu.SemaphoreType.DMA,
        ],
    )
    def gather_kernel(tbl_hbm, idx_hbm, out_hbm, idx_v, rows_v, sem):
        wid = lax.axis_index("s") * nc + lax.axis_index("c")
        tile_base = wid * b_per_w
        for c in range(b_per_w // chunk):
            base = tile_base + c * chunk
            pltpu.sync_copy(idx_hbm.at[pl.ds(base, chunk)], idx_v)
            pltpu.async_copy(tbl_hbm.at[idx_v], rows_v, sem).wait()
            pltpu.sync_copy(rows_v, out_hbm.at[pl.ds(base, chunk)])

    return gather_kernel(pairs, pidx)


def _mlp_body(uw_ref, iw_ref, up_ref, ip_ref, low_ref, w1u_ref, w1i_ref,
              b1_ref, w2_ref, b2_ref, out_ref):
    low = low_ref[...]
    flip = 1.0 - 2.0 * low
    umask = low + up_ref[...] * flip
    imask = low + ip_ref[...] * flip
    uwm = uw_ref[...] * umask
    iwm = iw_ref[...] * imask
    h = (
        jnp.dot(uwm, w1u_ref[...], preferred_element_type=jnp.float32)
        + jnp.dot(iwm, w1i_ref[...], preferred_element_type=jnp.float32)
        + b1_ref[...]
    )
    h = jnp.maximum(h, 0.0)
    out_ref[...] = jnp.sum(h * w2_ref[...], axis=1, keepdims=True) + b2_ref[...]


def _tc_mlp(uw, iw, up, ip, W1, b1, W2, b2, batch, block):
    width = uw.shape[1]
    half = width // 2
    hidden = W1.shape[1]
    w1u = jnp.concatenate([W1[:half], W1[:half]], axis=0)
    w1i = jnp.concatenate([W1[half:], W1[half:]], axis=0)
    b1r = b1.reshape(1, hidden)
    w2r = W2.reshape(1, hidden)
    b2r = b2.reshape(1, 1)
    low = jnp.concatenate(
        [jnp.ones((1, half), jnp.float32), jnp.zeros((1, half), jnp.float32)],
        axis=1,
    )
    grid = (batch // block,)
    return pl.pallas_call(
        _mlp_body,
        grid=grid,
        in_specs=[
            pl.BlockSpec((block, width), lambda i: (i, 0)),
            pl.BlockSpec((block, width), lambda i: (i, 0)),
            pl.BlockSpec((block, 1), lambda i: (i, 0)),
            pl.BlockSpec((block, 1), lambda i: (i, 0)),
            pl.BlockSpec((1, width), lambda i: (0, 0)),
            pl.BlockSpec((width, hidden), lambda i: (0, 0)),
            pl.BlockSpec((width, hidden), lambda i: (0, 0)),
            pl.BlockSpec((1, hidden), lambda i: (0, 0)),
            pl.BlockSpec((1, hidden), lambda i: (0, 0)),
            pl.BlockSpec((1, 1), lambda i: (0, 0)),
        ],
        out_specs=pl.BlockSpec((block, 1), lambda i: (i, 0)),
        out_shape=jax.ShapeDtypeStruct((batch, 1), jnp.float32),
    )(uw, iw, up, ip, low, w1u, w1i, b1r, w2r, b2r)


@jax.jit
def kernel(user_idx, item_idx, user_table, item_table, W1, b1, W2, b2):
    batch = user_idx.shape[0]
    ui = user_idx.astype(jnp.int32)
    ii = item_idx.astype(jnp.int32)
    upidx, up = _pair_index(ui)
    ipidx, ip = _pair_index(ii)
    item_pairs = _tc_repack_pairs(item_table.T)
    iw = _sc_pair_gather(item_pairs, ipidx, batch)
    user_pairs = _tc_repack_pairs(user_table.T)
    uw = _sc_pair_gather(user_pairs, upidx, batch)
    upf = up.astype(jnp.float32).reshape(batch, 1)
    ipf = ip.astype(jnp.float32).reshape(batch, 1)
    return _tc_mlp(uw, iw, upf, ipf, W1, b1, W2, b2, batch, block=2048)


# speedup vs baseline: 2.3219x; 1.0343x over previous
import functools

import jax
import jax.numpy as jnp
from jax import lax
from jax.experimental import pallas as pl
from jax.experimental.pallas import tpu as pltpu
from jax.experimental.pallas import tpu_sc as plsc

_COLS = 32768


def _repack_body(in_ref, ident_ref, out_ref):
    t = lax.dot_general(in_ref[...], ident_ref[...],
                        (((0,), (0,)), ((), ())),
                        preferred_element_type=jnp.float32)
    half = t.shape[0] // 2
    out_ref[...] = jnp.concatenate([t[:half], t[half:]], axis=1)


def _tc_repack_pairs(tbl_t):
    embed = tbl_t.shape[0]
    n = tbl_t.shape[1]
    n_blocks = pl.cdiv(n, _COLS)
    ident = jnp.eye(embed, dtype=jnp.float32)
    return pl.pallas_call(
        _repack_body,
        grid=(n_blocks,),
        compiler_params=pltpu.CompilerParams(
            dimension_semantics=("parallel",)),
        in_specs=[pl.BlockSpec((embed, _COLS), lambda j: (0, j)),
                  pl.BlockSpec((embed, embed), lambda j: (0, 0))],
        out_specs=pl.BlockSpec((_COLS // 2, 2 * embed), lambda j: (j, 0)),
        out_shape=jax.ShapeDtypeStruct(
            (n_blocks * _COLS // 2, 2 * embed), jnp.float32),
    )(tbl_t, ident)


def _pair_index(r):
    half_cols = _COLS // 2
    pidx = (r // _COLS) * half_cols + lax.rem(r, half_cols)
    p = lax.rem(r // half_cols, 2)
    return pidx, p


def _sc_pair_gather(pairs, pidx, batch):
    info = plsc.get_sparse_core_info()
    nc, ns = info.num_cores, info.num_subcores
    nw = nc * ns
    b_per_w = batch // nw
    chunk = b_per_w // 2
    width = pairs.shape[1]
    mesh = plsc.VectorSubcoreMesh(core_axis_name="c", subcore_axis_name="s")

    @functools.partial(
        pl.kernel,
        mesh=mesh,
        out_type=jax.ShapeDtypeStruct((batch, width), jnp.float32),
        scratch_types=[
            pltpu.VMEM((chunk,), jnp.int32),
            pltpu.VMEM((chunk, width), jnp.float32),
            pltpu.SemaphoreType.DMA,
        ],
    )
    def gather_kernel(tbl_hbm, idx_hbm, out_hbm, idx_v, rows_v, sem):
        wid = lax.axis_index("s") * nc + lax.axis_index("c")
        tile_base = wid * b_per_w
        for c in range(b_per_w // chunk):
            base = tile_base + c * chunk
            pltpu.sync_copy(idx_hbm.at[pl.ds(base, chunk)], idx_v)
            pltpu.async_copy(tbl_hbm.at[idx_v], rows_v, sem).wait()
            pltpu.sync_copy(rows_v, out_hbm.at[pl.ds(base, chunk)])

    return gather_kernel(pairs, pidx)


def _mlp_body(uw_ref, iw_ref, up_ref, ip_ref, low_ref, w1u_ref, w1i_ref,
              b1_ref, w2_ref, b2_ref, out_ref):
    low = low_ref[...]
    flip = 1.0 - 2.0 * low
    umask = low + up_ref[...] * flip
    imask = low + ip_ref[...] * flip
    uwm = uw_ref[...] * umask
    iwm = iw_ref[...] * imask
    h = (
        jnp.dot(uwm, w1u_ref[...], preferred_element_type=jnp.float32)
        + jnp.dot(iwm, w1i_ref[...], preferred_element_type=jnp.float32)
        + b1_ref[...]
    )
    h = jnp.maximum(h, 0.0)
    out_ref[...] = jnp.sum(h * w2_ref[...], axis=1, keepdims=True) + b2_ref[...]


def _tc_mlp(uw, iw, up, ip, W1, b1, W2, b2, batch, block):
    width = uw.shape[1]
    half = width // 2
    hidden = W1.shape[1]
    w1u = jnp.concatenate([W1[:half], W1[:half]], axis=0)
    w1i = jnp.concatenate([W1[half:], W1[half:]], axis=0)
    b1r = b1.reshape(1, hidden)
    w2r = W2.reshape(1, hidden)
    b2r = b2.reshape(1, 1)
    low = jnp.concatenate(
        [jnp.ones((1, half), jnp.float32), jnp.zeros((1, half), jnp.float32)],
        axis=1,
    )
    grid = (batch // block,)
    return pl.pallas_call(
        _mlp_body,
        grid=grid,
        in_specs=[
            pl.BlockSpec((block, width), lambda i: (i, 0)),
            pl.BlockSpec((block, width), lambda i: (i, 0)),
            pl.BlockSpec((block, 1), lambda i: (i, 0)),
            pl.BlockSpec((block, 1), lambda i: (i, 0)),
            pl.BlockSpec((1, width), lambda i: (0, 0)),
            pl.BlockSpec((width, hidden), lambda i: (0, 0)),
            pl.BlockSpec((width, hidden), lambda i: (0, 0)),
            pl.BlockSpec((1, hidden), lambda i: (0, 0)),
            pl.BlockSpec((1, hidden), lambda i: (0, 0)),
            pl.BlockSpec((1, 1), lambda i: (0, 0)),
        ],
        out_specs=pl.BlockSpec((block, 1), lambda i: (i, 0)),
        out_shape=jax.ShapeDtypeStruct((batch, 1), jnp.float32),
    )(uw, iw, up, ip, low, w1u, w1i, b1r, w2r, b2r)


@jax.jit
def kernel(user_idx, item_idx, user_table, item_table, W1, b1, W2, b2):
    batch = user_idx.shape[0]
    ui = user_idx.astype(jnp.int32)
    ii = item_idx.astype(jnp.int32)
    upidx, up = _pair_index(ui)
    ipidx, ip = _pair_index(ii)
    item_pairs = _tc_repack_pairs(item_table.T)
    iw = _sc_pair_gather(item_pairs, ipidx, batch)
    user_pairs = _tc_repack_pairs(user_table.T)
    uw = _sc_pair_gather(user_pairs, upidx, batch)
    upf = up.astype(jnp.float32).reshape(batch, 1)
    ipf = ip.astype(jnp.float32).reshape(batch, 1)
    return _tc_mlp(uw, iw, upf, ipf, W1, b1, W2, b2, batch, block=2048)


# speedup vs baseline: 2.3350x; 1.0056x over previous
import functools

import jax
import jax.numpy as jnp
from jax import lax
from jax.experimental import pallas as pl
from jax.experimental.pallas import tpu as pltpu
from jax.experimental.pallas import tpu_sc as plsc

_COLS = 32768


def _repack_body(in_ref, ident_ref, out_ref):
    embed, cols = in_ref.shape
    half = cols // 2
    t_low = lax.dot_general(in_ref[:, :half], ident_ref[...],
                            (((0,), (0,)), ((), ())),
                            preferred_element_type=jnp.float32)
    t_high = lax.dot_general(in_ref[:, half:], ident_ref[...],
                             (((0,), (0,)), ((), ())),
                             preferred_element_type=jnp.float32)
    out_ref[:, :embed] = t_low
    out_ref[:, embed:] = t_high


def _tc_repack_pairs(tbl_t):
    embed = tbl_t.shape[0]
    n = tbl_t.shape[1]
    n_blocks = pl.cdiv(n, _COLS)
    ident = jnp.eye(embed, dtype=jnp.float32)
    return pl.pallas_call(
        _repack_body,
        grid=(n_blocks,),
        compiler_params=pltpu.CompilerParams(
            dimension_semantics=("parallel",),
            fuse_transposed_lhs_in_matmul=True),
        in_specs=[pl.BlockSpec((embed, _COLS), lambda j: (0, j)),
                  pl.BlockSpec((embed, embed), lambda j: (0, 0))],
        out_specs=pl.BlockSpec((_COLS // 2, 2 * embed), lambda j: (j, 0)),
        out_shape=jax.ShapeDtypeStruct(
            (n_blocks * _COLS // 2, 2 * embed), jnp.float32),
    )(tbl_t, ident)


def _pair_index(r):
    half_cols = _COLS // 2
    pidx = (r // _COLS) * half_cols + lax.rem(r, half_cols)
    p = lax.rem(r // half_cols, 2)
    return pidx, p


def _sc_pair_gather(pairs, pidx, batch):
    info = plsc.get_sparse_core_info()
    nc, ns = info.num_cores, info.num_subcores
    nw = nc * ns
    b_per_w = batch // nw
    chunk = b_per_w // 2
    width = pairs.shape[1]
    mesh = plsc.VectorSubcoreMesh(core_axis_name="c", subcore_axis_name="s")

    @functools.partial(
        pl.kernel,
        mesh=mesh,
        out_type=jax.ShapeDtypeStruct((batch, width), jnp.float32),
        scratch_types=[
            pltpu.VMEM((chunk,), jnp.int32),
            pltpu.VMEM((chunk, width), jnp.float32),
            pltpu.SemaphoreType.DMA,
        ],
    )
    def gather_kernel(tbl_hbm, idx_hbm, out_hbm, idx_v, rows_v, sem):
        wid = lax.axis_index("s") * nc + lax.axis_index("c")
        tile_base = wid * b_per_w
        for c in range(b_per_w // chunk):
            base = tile_base + c * chunk
            pltpu.sync_copy(idx_hbm.at[pl.ds(base, chunk)], idx_v)
            pltpu.async_copy(tbl_hbm.at[idx_v], rows_v, sem).wait()
            pltpu.sync_copy(rows_v, out_hbm.at[pl.ds(base, chunk)])

    return gather_kernel(pairs, pidx)


def _mlp_body(uw_ref, iw_ref, up_ref, ip_ref, low_ref, w1u_ref, w1i_ref,
              b1_ref, w2_ref, b2_ref, out_ref):
    low = low_ref[...]
    flip = 1.0 - 2.0 * low
    umask = low + up_ref[...] * flip
    imask = low + ip_ref[...] * flip
    uwm = uw_ref[...] * umask
    iwm = iw_ref[...] * imask
    h = (
        jnp.dot(uwm, w1u_ref[...], preferred_element_type=jnp.float32)
        + jnp.dot(iwm, w1i_ref[...], preferred_element_type=jnp.float32)
        + b1_ref[...]
    )
    h = jnp.maximum(h, 0.0)
    out_ref[...] = jnp.sum(h * w2_ref[...], axis=1, keepdims=True) + b2_ref[...]


def _tc_mlp(uw, iw, up, ip, W1, b1, W2, b2, batch, block):
    width = uw.shape[1]
    half = width // 2
    hidden = W1.shape[1]
    w1u = jnp.concatenate([W1[:half], W1[:half]], axis=0)
    w1i = jnp.concatenate([W1[half:], W1[half:]], axis=0)
    b1r = b1.reshape(1, hidden)
    w2r = W2.reshape(1, hidden)
    b2r = b2.reshape(1, 1)
    low = jnp.concatenate(
        [jnp.ones((1, half), jnp.float32), jnp.zeros((1, half), jnp.float32)],
        axis=1,
    )
    grid = (batch // block,)
    return pl.pallas_call(
        _mlp_body,
        grid=grid,
        in_specs=[
            pl.BlockSpec((block, width), lambda i: (i, 0)),
            pl.BlockSpec((block, width), lambda i: (i, 0)),
            pl.BlockSpec((block, 1), lambda i: (i, 0)),
            pl.BlockSpec((block, 1), lambda i: (i, 0)),
            pl.BlockSpec((1, width), lambda i: (0, 0)),
            pl.BlockSpec((width, hidden), lambda i: (0, 0)),
            pl.BlockSpec((width, hidden), lambda i: (0, 0)),
            pl.BlockSpec((1, hidden), lambda i: (0, 0)),
            pl.BlockSpec((1, hidden), lambda i: (0, 0)),
            pl.BlockSpec((1, 1), lambda i: (0, 0)),
        ],
        out_specs=pl.BlockSpec((block, 1), lambda i: (i, 0)),
        out_shape=jax.ShapeDtypeStruct((batch, 1), jnp.float32),
    )(uw, iw, up, ip, low, w1u, w1i, b1r, w2r, b2r)


@jax.jit
def kernel(user_idx, item_idx, user_table, item_table, W1, b1, W2, b2):
    batch = user_idx.shape[0]
    ui = user_idx.astype(jnp.int32)
    ii = item_idx.astype(jnp.int32)
    upidx, up = _pair_index(ui)
    ipidx, ip = _pair_index(ii)
    item_pairs = _tc_repack_pairs(item_table.T)
    iw = _sc_pair_gather(item_pairs, ipidx, batch)
    user_table, item_pairs = lax.optimization_barrier((user_table, item_pairs))
    user_pairs = _tc_repack_pairs(user_table.T)
    uw = _sc_pair_gather(user_pairs, upidx, batch)
    upf = up.astype(jnp.float32).reshape(batch, 1)
    ipf = ip.astype(jnp.float32).reshape(batch, 1)
    return _tc_mlp(uw, iw, upf, ipf, W1, b1, W2, b2, batch, block=2048)
